# baseline (speedup 1.0000x reference)
import jax
import jax.numpy as jnp
from jax.experimental import pallas as pl


def _gat_layer(x, src, dst, W, a_src, a_dst, b):
    n = x.shape[0]
    s = x @ (W @ a_src)
    t = x @ (W @ a_dst)
    e = jax.nn.leaky_relu(s[src] + t[dst], negative_slope=0.2)
    w = jnp.exp(e)
    den = jax.ops.segment_sum(w, dst, num_segments=n)
    agg = jax.ops.segment_sum(w[:, None] * x[src], dst, num_segments=n)
    return (agg / (den + 1e-16)[:, None]) @ W + b


def _mlp_kernel(h_ref, wm1_ref, bm1_ref, wm2_ref, bm2_ref, out_ref):
    h = h_ref[...]
    z = jnp.maximum(jnp.dot(h, wm1_ref[...], preferred_element_type=jnp.float32)
                    + bm1_ref[...][None, :], 0.0)
    z = jnp.dot(z, wm2_ref[...], preferred_element_type=jnp.float32) + bm2_ref[...][None, :]
    z = z - jnp.max(z, axis=-1, keepdims=True)
    ez = jnp.exp(z)
    out_ref[...] = ez / jnp.sum(ez, axis=-1, keepdims=True)


def kernel(x, edge_index, doc_map, W0, a_src0, a_dst0, b0, W1, a_src1, a_dst1, b1, Wm1, bm1, Wm2, bm2):
    src, dst = edge_index[0], edge_index[1]
    h = jax.nn.relu(_gat_layer(x, src, dst, W0, a_src0, a_dst0, b0))
    h = jax.nn.relu(_gat_layer(h, src, dst, W1, a_src1, a_dst1, b1))
    hd = h[doc_map]
    return pl.pallas_call(
        _mlp_kernel,
        out_shape=jax.ShapeDtypeStruct((hd.shape[0], Wm2.shape[1]), jnp.float32),
    )(hd, Wm1, bm1, Wm2, bm2)


# speedup vs baseline: 12.4901x; 12.4901x over previous
import jax
import jax.numpy as jnp
from jax import lax
from jax.experimental import pallas as pl
from jax.experimental.pallas import tpu as pltpu
from jax.experimental.pallas import tpu_sc as plsc

N = 10000
D = 128
E = 320000
NC = 2
NS = 16
HF = D // NC
ES = E // NS
CH = 80
NCH = ES // CH
RPT = 640
NPAD = RPT * NS
LANES = 16


def _sc_agg_body(x2_hbm, src_hbm, dst_hbm, s2_hbm, t_hbm,
                 agg_out, den_out,
                 sval2_v, tval_v, src_v, dst_v, srcx_v, wc_v, den_v, rows_v,
                 agg_sh, gsem, ssem):
    cid = lax.axis_index("c")
    sid = lax.axis_index("s")

    pltpu.sync_copy(s2_hbm, sval2_v)
    pltpu.sync_copy(t_hbm, tval_v)
    pltpu.sync_copy(src_hbm.at[sid], src_v)
    pltpu.sync_copy(dst_hbm.at[sid], dst_v)

    def _zrows(j, _):
        for f in range(HF // LANES):
            rows_v[j, pl.ds(f * LANES, LANES)] = jnp.zeros((LANES,), jnp.float32)
        return 0
    lax.fori_loop(0, CH, _zrows, 0)
    for z in range(RPT // CH):
        pltpu.sync_copy(rows_v, agg_sh.at[pl.ds(sid * RPT + z * CH, CH)])

    def _zden(i, _):
        den_v[pl.ds(i * LANES, LANES)] = jnp.zeros((LANES,), jnp.float32)
        return 0
    lax.fori_loop(0, N // LANES, _zden, 0)

    plsc.subcore_barrier()

    def _chunk(c, _):
        def _p1(j, _):
            sidx2 = src_v[c, pl.ds(j * LANES, LANES)]
            didx = dst_v[c, pl.ds(j * LANES, LANES)]
            sg = plsc.load_gather(sval2_v, [sidx2])
            tg = plsc.load_gather(tval_v, [didx])
            z = sg + tg
            w = jnp.exp(jnp.maximum(z, 0.2 * z))
            wc_v[pl.ds(j * LANES, LANES)] = w
            plsc.addupdate_scatter(den_v, [didx], w)
            srcx_v[pl.ds(j * LANES, LANES)] = sidx2 + cid
            return 0
        lax.fori_loop(0, CH // LANES, _p1, 0)

        pltpu.async_copy(x2_hbm.at[srcx_v], rows_v, gsem).wait()

        def _scale(j, _):
            wb = plsc.load_gather(wc_v, [jnp.full((LANES,), j, jnp.int32)])
            for f in range(HF // LANES):
                rows_v[j, pl.ds(f * LANES, LANES)] = rows_v[j, pl.ds(f * LANES, LANES)] * wb
            return 0
        lax.fori_loop(0, CH, _scale, 0)

        pltpu.async_copy(rows_v, agg_sh.at[dst_v.at[c]], ssem, add=True).wait()
        return 0
    lax.fori_loop(0, NCH, _chunk, 0)

    plsc.subcore_barrier()
    pltpu.sync_copy(agg_sh.at[pl.ds(sid * RPT, RPT)],
                    agg_out.at[cid, pl.ds(sid * RPT, RPT)])

    @pl.when(cid == 0)
    def _():
        pltpu.sync_copy(den_v, den_out.at[sid])


_sc_agg = pl.kernel(
    _sc_agg_body,
    out_type=[jax.ShapeDtypeStruct((NC, NPAD, HF), jnp.float32),
              jax.ShapeDtypeStruct((NS, N), jnp.float32)],
    mesh=plsc.VectorSubcoreMesh(core_axis_name="c", subcore_axis_name="s"),
    compiler_params=pltpu.CompilerParams(needs_layout_passes=False,
                                         use_tc_tiling_on_sc=False),
    scratch_types=[
        pltpu.VMEM((2 * N,), jnp.float32),
        pltpu.VMEM((N,), jnp.float32),
        pltpu.VMEM((NCH, CH), jnp.int32),
        pltpu.VMEM((NCH, CH), jnp.int32),
        pltpu.VMEM((CH,), jnp.int32),
        pltpu.VMEM((CH,), jnp.float32),
        pltpu.VMEM((N,), jnp.float32),
        pltpu.VMEM((CH, HF), jnp.float32),
        pltpu.VMEM_SHARED((NPAD, HF), jnp.float32),
        pltpu.SemaphoreType.DMA,
        pltpu.SemaphoreType.DMA,
    ],
)


def _st_tc(x_ref, w_ref, av_ref, out_ref):
    uv = jnp.dot(w_ref[...], av_ref[...], preferred_element_type=jnp.float32)
    out_ref[...] = jnp.dot(x_ref[...], uv, preferred_element_type=jnp.float32)


def _mid_tc(a0_ref, a1_ref, denp_ref, w0_ref, b0_ref, w1_ref, av1_ref, h1_ref, st1_ref):
    den = jnp.sum(denp_ref[...], axis=0) + 1e-16
    h1 = (jnp.dot(a0_ref[...] / den[:, None], w0_ref[0:HF, :],
                  preferred_element_type=jnp.float32)
          + jnp.dot(a1_ref[...] / den[:, None], w0_ref[HF:D, :],
                    preferred_element_type=jnp.float32)
          + b0_ref[...][None, :])
    h1 = jnp.maximum(h1, 0.0)
    h1_ref[...] = h1
    uv1 = jnp.dot(w1_ref[...], av1_ref[...], preferred_element_type=jnp.float32)
    st1_ref[...] = jnp.dot(h1, uv1, preferred_element_type=jnp.float32)


def _head_tc(a0_ref, a1_ref, denp_ref, w1_ref, b1_ref, wm1_ref, bm1_ref, wm2_ref, bm2_ref, out_ref):
    den = jnp.sum(denp_ref[...], axis=0) + 1e-16
    h = (jnp.dot(a0_ref[...] / den[:, None], w1_ref[0:HF, :],
                 preferred_element_type=jnp.float32)
         + jnp.dot(a1_ref[...] / den[:, None], w1_ref[HF:D, :],
                   preferred_element_type=jnp.float32)
         + b1_ref[...][None, :])
    h = jnp.maximum(h, 0.0)
    z = jnp.maximum(
        jnp.dot(h, wm1_ref[...], preferred_element_type=jnp.float32) + bm1_ref[...][None, :],
        0.0)
    z = jnp.dot(z, wm2_ref[...], preferred_element_type=jnp.float32) + bm2_ref[...][None, :]
    z = z - jnp.max(z, axis=-1, keepdims=True)
    ez = jnp.exp(z)
    out_ref[...] = ez / jnp.sum(ez, axis=-1, keepdims=True)


def kernel(x, edge_index, doc_map, W0, a_src0, a_dst0, b0, W1, a_src1, a_dst1, b1, Wm1, bm1, Wm2, bm2):
    src2 = (edge_index[0] * 2).reshape(NS, NCH, CH)
    dst2 = edge_index[1].reshape(NS, NCH, CH)
    pad = jnp.zeros((D, 6), jnp.float32)
    av0 = jnp.concatenate([a_src0[:, None], a_dst0[:, None], pad], axis=1)
    av1 = jnp.concatenate([a_src1[:, None], a_dst1[:, None], pad], axis=1)

    st0 = pl.pallas_call(
        _st_tc,
        out_shape=jax.ShapeDtypeStruct((N, 8), jnp.float32),
    )(x, W0, av0)

    aggp, denp = _sc_agg(x.reshape(2 * N, HF), src2, dst2,
                         jnp.repeat(st0[:, 0], 2), st0[:, 1])

    h1, st1 = pl.pallas_call(
        _mid_tc,
        out_shape=[jax.ShapeDtypeStruct((N, D), jnp.float32),
                   jax.ShapeDtypeStruct((N, 8), jnp.float32)],
    )(aggp[0, :N], aggp[1, :N], denp, W0, b0, W1, av1)

    aggp1, denp1 = _sc_agg(h1.reshape(2 * N, HF), src2, dst2,
                           jnp.repeat(st1[:, 0], 2), st1[:, 1])

    a0d = aggp1[0, doc_map, :]
    a1d = aggp1[1, doc_map, :]
    dend = denp1[:, doc_map]
    return pl.pallas_call(
        _head_tc,
        out_shape=jax.ShapeDtypeStruct((doc_map.shape[0], Wm2.shape[1]), jnp.float32),
    )(a0d, a1d, dend, W1, b1, Wm1, bm1, Wm2, bm2)


# speedup vs baseline: 38.1446x; 3.0540x over previous
import jax
import jax.numpy as jnp
from jax import lax
from jax.experimental import pallas as pl
from jax.experimental.pallas import tpu as pltpu
from jax.experimental.pallas import tpu_sc as plsc

N = 10000
D = 128
E = 320000
NC = 2
NS = 16
HF = D // NC
ES = E // NS
CH = 80
NCH = ES // CH
RPT = 640
NPAD = RPT * NS
LANES = 16


SLOTS = 5
SHIFT = 3
NR = NCH // SLOTS


def _sc_agg_body(x2_hbm, src_hbm, dst_hbm, s_hbm, t_hbm,
                 agg_out, den_out,
                 sval_v, tval_v, src_v, dst_v,
                 rows0, rows1, rows2, rows3, rows4,
                 wc0, wc1, wc2, wc3, wc4,
                 sx0, sx1, sx2, sx3, sx4,
                 agg_sh, den_sh,
                 gs0, gs1, gs2, gs3, gs4, ss0, ss1, ss2, ss3, ss4):
    cid = lax.axis_index("c")
    sid = lax.axis_index("s")
    rows = [rows0, rows1, rows2, rows3, rows4]
    wc = [wc0, wc1, wc2, wc3, wc4]
    sx = [sx0, sx1, sx2, sx3, sx4]
    gsem = [gs0, gs1, gs2, gs3, gs4]
    ssem = [ss0, ss1, ss2, ss3, ss4]

    pltpu.sync_copy(s_hbm, sval_v)
    pltpu.sync_copy(t_hbm, tval_v)
    pltpu.sync_copy(src_hbm.at[sid], src_v)
    pltpu.sync_copy(dst_hbm.at[sid], dst_v)

    def _zrows(j, _):
        for f in range(HF // LANES):
            rows0[j, pl.ds(f * LANES, LANES)] = jnp.zeros((LANES,), jnp.float32)
        return 0
    lax.fori_loop(0, CH, _zrows, 0)
    def _zwc(j, _):
        wc0[pl.ds(j * LANES, LANES)] = jnp.zeros((LANES,), jnp.float32)
        return 0
    lax.fori_loop(0, CH // LANES, _zwc, 0)
    for z in range(RPT // CH):
        pltpu.sync_copy(rows0, agg_sh.at[pl.ds(sid * RPT + z * CH, CH)])
        pltpu.sync_copy(wc0, den_sh.at[pl.ds(sid * RPT + z * CH, CH)])

    plsc.subcore_barrier()

    def _p1(cp, k):
        def _p1j(j, _):
            sl = pl.ds(j * LANES, LANES)
            sidx2 = src_v[cp, sl]
            didx = dst_v[cp, sl]
            sg = plsc.load_gather(sval_v, [lax.shift_right_logical(sidx2, 1)])
            tg = plsc.load_gather(tval_v, [didx])
            z = sg + tg
            w = jnp.exp(jnp.maximum(z, 0.2 * z))
            wc[k][sl] = w
            sx[k][sl] = sidx2 + cid
            return 0
        lax.fori_loop(0, CH // LANES, _p1j, 0)

    def _den_start(cp, k):
        pltpu.async_copy(wc[k], den_sh.at[dst_v.at[cp]], ssem[k], add=True)

    def _den_wait(cp, k):
        pltpu.make_async_copy(wc[k], den_sh.at[dst_v.at[cp]], ssem[k]).wait()

    def _g_start(cp, k):
        pltpu.async_copy(x2_hbm.at[sx[k]], rows[k], gsem[k])

    def _g_wait(cp, k):
        pltpu.make_async_copy(x2_hbm.at[sx[k]], rows[k], gsem[k]).wait()

    def _s_start(c, k):
        pltpu.async_copy(rows[k], agg_sh.at[dst_v.at[c]], ssem[k], add=True)

    def _s_wait(c, k):
        pltpu.make_async_copy(rows[k], agg_sh.at[dst_v.at[c]], ssem[k]).wait()

    def _prefetch(cp, k, drain):
        if drain:
            _s_wait(cp - SLOTS, k)
            _den_wait(cp - SLOTS, k)
        _p1(cp, k)
        _den_start(cp, k)
        _g_start(cp, k)

    for k in range(SHIFT):
        _prefetch(jnp.int32(k), k, drain=False)

    def _round(p, _):
        c0 = SLOTS * p
        for k in range(SLOTS):
            c = c0 + k
            _g_wait(c, k)

            @plsc.parallel_loop(0, CH, 1, unroll=2)
            def _scale(j):
                wb = plsc.load_gather(wc[k], [jnp.full((LANES,), j, jnp.int32)])
                for f in range(HF // LANES):
                    rows[k][j, pl.ds(f * LANES, LANES)] = (
                        rows[k][j, pl.ds(f * LANES, LANES)] * wb)

            _s_start(c, k)

            cp = c + SHIFT
            j3 = (k + SHIFT) % SLOTS

            @pl.when(cp < NCH)
            def _():
                @pl.when(cp >= SLOTS)
                def _():
                    _s_wait(cp - SLOTS, j3)
                    _den_wait(cp - SLOTS, j3)
                _p1(cp, j3)
                _den_start(cp, j3)
                _g_start(cp, j3)
        return 0
    lax.fori_loop(0, NR, _round, 0)

    for k in range(SLOTS):
        _s_wait(jnp.int32(NCH - SLOTS + k), k)
        _den_wait(jnp.int32(NCH - SLOTS + k), k)

    plsc.subcore_barrier()
    pltpu.sync_copy(agg_sh.at[pl.ds(sid * RPT, RPT)],
                    agg_out.at[cid, pl.ds(sid * RPT, RPT)])

    @pl.when(cid == 0)
    def _():
        pltpu.sync_copy(den_sh.at[pl.ds(sid * RPT, RPT)],
                        den_out.at[pl.ds(sid * RPT, RPT)])


_sc_agg = pl.kernel(
    _sc_agg_body,
    out_type=[jax.ShapeDtypeStruct((NC, NPAD, HF), jnp.float32),
              jax.ShapeDtypeStruct((NPAD,), jnp.float32)],
    mesh=plsc.VectorSubcoreMesh(core_axis_name="c", subcore_axis_name="s"),
    compiler_params=pltpu.CompilerParams(needs_layout_passes=False,
                                         use_tc_tiling_on_sc=False),
    scratch_types=[
        pltpu.VMEM((N,), jnp.float32),
        pltpu.VMEM((N,), jnp.float32),
        pltpu.VMEM((NCH, CH), jnp.int32),
        pltpu.VMEM((NCH, CH), jnp.int32),
    ] + [pltpu.VMEM((CH, HF), jnp.float32) for _ in range(SLOTS)]
      + [pltpu.VMEM((CH,), jnp.float32) for _ in range(SLOTS)]
      + [pltpu.VMEM((CH,), jnp.int32) for _ in range(SLOTS)]
      + [pltpu.VMEM_SHARED((NPAD, HF), jnp.float32),
         pltpu.VMEM_SHARED((NPAD,), jnp.float32)]
      + [pltpu.SemaphoreType.DMA] * (2 * SLOTS),
)


def _st_tc(x_ref, w_ref, av_ref, out_ref):
    uv = jnp.dot(w_ref[...], av_ref[...], preferred_element_type=jnp.float32)
    out_ref[...] = jnp.dot(x_ref[...], uv, preferred_element_type=jnp.float32)


def _mid_tc(a0_ref, a1_ref, den_ref, w0_ref, b0_ref, w1_ref, av1_ref, h1_ref, st1_ref):
    den = den_ref[...] + 1e-16
    h1 = (jnp.dot(a0_ref[...] / den[:, None], w0_ref[0:HF, :],
                  preferred_element_type=jnp.float32)
          + jnp.dot(a1_ref[...] / den[:, None], w0_ref[HF:D, :],
                    preferred_element_type=jnp.float32)
          + b0_ref[...][None, :])
    h1 = jnp.maximum(h1, 0.0)
    h1_ref[...] = h1
    uv1 = jnp.dot(w1_ref[...], av1_ref[...], preferred_element_type=jnp.float32)
    st1_ref[...] = jnp.dot(h1, uv1, preferred_element_type=jnp.float32)


def _head_tc(a0_ref, a1_ref, den_ref, w1_ref, b1_ref, wm1_ref, bm1_ref, wm2_ref, bm2_ref, out_ref):
    den = den_ref[...] + 1e-16
    h = (jnp.dot(a0_ref[...] / den[:, None], w1_ref[0:HF, :],
                 preferred_element_type=jnp.float32)
         + jnp.dot(a1_ref[...] / den[:, None], w1_ref[HF:D, :],
                   preferred_element_type=jnp.float32)
         + b1_ref[...][None, :])
    h = jnp.maximum(h, 0.0)
    z = jnp.maximum(
        jnp.dot(h, wm1_ref[...], preferred_element_type=jnp.float32) + bm1_ref[...][None, :],
        0.0)
    z = jnp.dot(z, wm2_ref[...], preferred_element_type=jnp.float32) + bm2_ref[...][None, :]
    z = z - jnp.max(z, axis=-1, keepdims=True)
    ez = jnp.exp(z)
    out_ref[...] = ez / jnp.sum(ez, axis=-1, keepdims=True)


def kernel(x, edge_index, doc_map, W0, a_src0, a_dst0, b0, W1, a_src1, a_dst1, b1, Wm1, bm1, Wm2, bm2):
    src2 = (edge_index[0] * 2).reshape(NS, NCH, CH)
    dst2 = edge_index[1].reshape(NS, NCH, CH)
    pad = jnp.zeros((D, 6), jnp.float32)
    av0 = jnp.concatenate([a_src0[:, None], a_dst0[:, None], pad], axis=1)
    av1 = jnp.concatenate([a_src1[:, None], a_dst1[:, None], pad], axis=1)

    st0 = pl.pallas_call(
        _st_tc,
        out_shape=jax.ShapeDtypeStruct((N, 8), jnp.float32),
    )(x, W0, av0)

    aggp, denp = _sc_agg(x.reshape(2 * N, HF), src2, dst2,
                         st0[:, 0], st0[:, 1])

    h1, st1 = pl.pallas_call(
        _mid_tc,
        out_shape=[jax.ShapeDtypeStruct((N, D), jnp.float32),
                   jax.ShapeDtypeStruct((N, 8), jnp.float32)],
    )(aggp[0, :N], aggp[1, :N], denp[:N], W0, b0, W1, av1)

    aggp1, denp1 = _sc_agg(h1.reshape(2 * N, HF), src2, dst2,
                           st1[:, 0], st1[:, 1])

    a0d = aggp1[0, doc_map, :]
    a1d = aggp1[1, doc_map, :]
    dend = denp1[doc_map]
    return pl.pallas_call(
        _head_tc,
        out_shape=jax.ShapeDtypeStruct((doc_map.shape[0], Wm2.shape[1]), jnp.float32),
    )(a0d, a1d, dend, W1, b1, Wm1, bm1, Wm2, bm2)
